# Initial kernel scaffold; baseline (speedup 1.0000x reference)
import jax
import jax.numpy as jnp
from jax.experimental import pallas as pl


def kernel(x, edge_index, batch, W1, b1, W2, b2, W3, b3, Wf, bf):
    raise NotImplementedError("write your pallas kernel here")


# speedup vs baseline: 12.0104x; 12.0104x over previous
import functools

import jax
import jax.numpy as jnp
from jax import lax
from jax.experimental import pallas as pl
from jax.experimental.pallas import tpu as pltpu
from jax.experimental.pallas import tpu_sc as plsc

NC = 2
NS = 16
NW = NC * NS
CH = 128


def _make_deg(n_pad, e_pad):
    epw = e_pad // NW
    n_chunks = epw // CH
    rpw = n_pad // NS
    mesh = plsc.VectorSubcoreMesh(core_axis_name="c", subcore_axis_name="s",
                                  num_cores=NC, num_subcores=NS)

    @functools.partial(
        pl.kernel,
        out_type=jax.ShapeDtypeStruct((NC, n_pad, 16), jnp.float32),
        mesh=mesh,
        scratch_types=[
            pltpu.VMEM((CH,), jnp.int32),
            pltpu.VMEM((CH, 16), jnp.float32),
            pltpu.VMEM_SHARED((n_pad, 16), jnp.float32),
        ],
        compiler_params=pltpu.CompilerParams(use_tc_tiling_on_sc=False),
    )
    def deg(dst_hbm, ones_hbm, zeros_hbm, out_hbm, didx, ones_v, acc):
        cid = lax.axis_index("c")
        sid = lax.axis_index("s")
        wid = sid * NC + cid
        pltpu.sync_copy(ones_hbm, ones_v)
        pltpu.sync_copy(zeros_hbm, acc.at[pl.ds(sid * rpw, rpw)])
        plsc.subcore_barrier()

        def body(i, carry):
            base = wid * epw + i * CH
            pltpu.sync_copy(dst_hbm.at[pl.ds(base, CH)], didx)
            pltpu.sync_copy(ones_v, acc.at[didx], add=True)
            return carry

        lax.fori_loop(0, n_chunks, body, 0)
        plsc.subcore_barrier()
        pltpu.sync_copy(acc.at[pl.ds(sid * rpw, rpw)],
                        out_hbm.at[cid, pl.ds(sid * rpw, rpw)])

    return deg


def _make_agg(n_pad, f, e_pad):
    epw = e_pad // NW
    n_chunks = epw // CH
    rpw = n_pad // NS
    mesh = plsc.VectorSubcoreMesh(core_axis_name="c", subcore_axis_name="s",
                                  num_cores=NC, num_subcores=NS)

    @functools.partial(
        pl.kernel,
        out_type=jax.ShapeDtypeStruct((NC, n_pad, f), jnp.float32),
        mesh=mesh,
        scratch_types=[
            pltpu.VMEM((CH,), jnp.int32),
            pltpu.VMEM((CH,), jnp.int32),
            pltpu.VMEM((CH, f), jnp.float32),
            pltpu.VMEM_SHARED((n_pad, f), jnp.float32),
            pltpu.SemaphoreType.DMA,
        ],
        compiler_params=pltpu.CompilerParams(use_tc_tiling_on_sc=False),
    )
    def agg(src_hbm, dst_hbm, g_hbm, zeros_hbm, out_hbm,
            sidx, didx, rows, acc, sem):
        cid = lax.axis_index("c")
        sid = lax.axis_index("s")
        wid = sid * NC + cid
        pltpu.sync_copy(zeros_hbm, acc.at[pl.ds(sid * rpw, rpw)])
        plsc.subcore_barrier()

        def body(i, carry):
            base = wid * epw + i * CH
            pltpu.sync_copy(src_hbm.at[pl.ds(base, CH)], sidx)
            pltpu.sync_copy(dst_hbm.at[pl.ds(base, CH)], didx)
            pltpu.async_copy(g_hbm.at[sidx], rows, sem).wait()
            pltpu.sync_copy(rows, acc.at[didx], add=True)
            return carry

        lax.fori_loop(0, n_chunks, body, 0)
        plsc.subcore_barrier()
        pltpu.sync_copy(acc.at[pl.ds(sid * rpw, rpw)],
                        out_hbm.at[cid, pl.ds(sid * rpw, rpw)])

    return agg


_R = 2000


def _dinv_of(deg_ref):
    deg = deg_ref[0, :, 0:1] + deg_ref[1, :, 0:1] + 1.0
    return lax.rsqrt(deg)


def _tc_prep1(x, w1, degp):
    n, fi = x.shape
    fo = w1.shape[1]
    nb = n // _R

    def body(x_ref, w_ref, deg_ref, out_ref):
        dinv = _dinv_of(deg_ref)
        out_ref[...] = jnp.dot(x_ref[...], w_ref[...],
                               preferred_element_type=jnp.float32) * dinv

    return pl.pallas_call(
        body,
        grid=(nb,),
        in_specs=[
            pl.BlockSpec((_R, fi), lambda i: (i, 0)),
            pl.BlockSpec((fi, fo), lambda i: (0, 0)),
            pl.BlockSpec((2, _R, 16), lambda i: (0, i, 0)),
        ],
        out_specs=pl.BlockSpec((_R, fo), lambda i: (i, 0)),
        out_shape=jax.ShapeDtypeStruct((n, fo), jnp.float32),
    )(x, w1, degp)


def _tc_mid(aggp, gprev, degp, b, w):
    n, fi = gprev.shape
    fo = w.shape[1]
    nb = n // _R

    def body(agg_ref, g_ref, deg_ref, b_ref, w_ref, out_ref):
        dinv = _dinv_of(deg_ref)
        h = dinv * (agg_ref[0] + agg_ref[1] + g_ref[...]) + b_ref[...]
        h = jnp.maximum(h, 0.0)
        out_ref[...] = jnp.dot(h, w_ref[...],
                               preferred_element_type=jnp.float32) * dinv

    return pl.pallas_call(
        body,
        grid=(nb,),
        in_specs=[
            pl.BlockSpec((2, _R, fi), lambda i: (0, i, 0)),
            pl.BlockSpec((_R, fi), lambda i: (i, 0)),
            pl.BlockSpec((2, _R, 16), lambda i: (0, i, 0)),
            pl.BlockSpec((1, fi), lambda i: (0, 0)),
            pl.BlockSpec((fi, fo), lambda i: (0, 0)),
        ],
        out_specs=pl.BlockSpec((_R, fo), lambda i: (i, 0)),
        out_shape=jax.ShapeDtypeStruct((n, fo), jnp.float32),
    )(aggp, gprev, degp, b, w)


def _tc_final(aggp, g3, degp, b3, wf, bf):
    n, fi = g3.shape
    nclass = wf.shape[1]
    nb = n // _R

    def body(agg_ref, g_ref, deg_ref, b_ref, wf_ref, bf_ref, out_ref, pool):
        i = pl.program_id(0)
        dinv = _dinv_of(deg_ref)
        h = dinv * (agg_ref[0] + agg_ref[1] + g_ref[...]) + b_ref[...]
        h = jnp.maximum(h, 0.0)
        m = jnp.max(h, axis=0, keepdims=True)

        @pl.when(i == 0)
        def _():
            pool[...] = m

        @pl.when(i > 0)
        def _():
            pool[...] = jnp.maximum(pool[...], m)

        @pl.when(i == nb - 1)
        def _():
            logits = jnp.dot(pool[...], wf_ref[...],
                             preferred_element_type=jnp.float32) + bf_ref[...]
            mx = jnp.max(logits, axis=1, keepdims=True)
            s = logits - mx
            out_ref[...] = s - jnp.log(jnp.sum(jnp.exp(s), axis=1,
                                               keepdims=True))

    return pl.pallas_call(
        body,
        grid=(nb,),
        in_specs=[
            pl.BlockSpec((2, _R, fi), lambda i: (0, i, 0)),
            pl.BlockSpec((_R, fi), lambda i: (i, 0)),
            pl.BlockSpec((2, _R, 16), lambda i: (0, i, 0)),
            pl.BlockSpec((1, fi), lambda i: (0, 0)),
            pl.BlockSpec((fi, nclass), lambda i: (0, 0)),
            pl.BlockSpec((1, nclass), lambda i: (0, 0)),
        ],
        out_specs=pl.BlockSpec((1, nclass), lambda i: (0, 0)),
        out_shape=jax.ShapeDtypeStruct((1, nclass), jnp.float32),
        scratch_shapes=[pltpu.VMEM((1, fi), jnp.float32)],
    )(aggp, g3, degp, b3, wf, bf)


def kernel(x, edge_index, batch, W1, b1, W2, b2, W3, b3, Wf, bf):
    n, _ = x.shape
    e = edge_index.shape[1]
    f1, f2, f3 = W1.shape[1], W2.shape[1], W3.shape[1]

    n_pad = ((n // 128) + 2) * 128
    e_pad = -(-e // (NW * CH)) * (NW * CH)
    rpw = n_pad // NS

    src = edge_index[0]
    dst = edge_index[1]
    pad_e = e_pad - e
    if pad_e:
        src = jnp.concatenate([src, jnp.zeros((pad_e,), src.dtype)])
        dst = jnp.concatenate([dst, jnp.full((pad_e,), n, dst.dtype)])

    ones16 = jnp.ones((CH, 16), jnp.float32)
    z16 = jnp.zeros((rpw, 16), jnp.float32)
    zf1 = jnp.zeros((rpw, f1), jnp.float32)
    zf2 = jnp.zeros((rpw, f2), jnp.float32)
    zf3 = jnp.zeros((rpw, f3), jnp.float32)

    degp = _make_deg(n_pad, e_pad)(dst, ones16, z16)

    g1 = _tc_prep1(x, W1, degp)
    a1 = _make_agg(n_pad, f1, e_pad)(src, dst, g1, zf1)
    g2 = _tc_mid(a1, g1, degp, b1.reshape(1, -1), W2)
    a2 = _make_agg(n_pad, f2, e_pad)(src, dst, g2, zf2)
    g3 = _tc_mid(a2, g2, degp, b2.reshape(1, -1), W3)
    a3 = _make_agg(n_pad, f3, e_pad)(src, dst, g3, zf3)
    return _tc_final(a3, g3, degp, b3.reshape(1, -1), Wf, bf.reshape(1, -1))
